# Initial kernel scaffold; baseline (speedup 1.0000x reference)
import functools

import jax
import jax.numpy as jnp
import numpy as np
from jax import lax
from jax.experimental import pallas as pl
from jax.experimental.pallas import tpu as pltpu
from jax.experimental.pallas import tpu_sc as plsc

N = 10000
E = 160000
D = 256
H = 8
C = 32

NS = 16
AROWS = 10240
RPS = AROWS // NS
W = 144
G = 80
EPW = E // NS
INV_SQRT_C = float(1.0 / np.sqrt(C))
INV_SQRT_H = float(1.0 / np.sqrt(H))

_MESH = plsc.VectorSubcoreMesh(core_axis_name="c", subcore_axis_name="s")
_f32 = jnp.float32


def _zero_rows(rowbuf, nrows, width):
    z = jnp.zeros((16,), _f32)

    @pl.loop(0, nrows)
    def _(r):
        for j in range(width // 16):
            rowbuf[r, pl.ds(j * 16, 16)] = z


def _sc_layer0(q, k, vf, src, dst, pm16):
    @functools.partial(
        pl.kernel,
        out_type=(
            jax.ShapeDtypeStruct((2 * AROWS, W), _f32),
            jax.ShapeDtypeStruct((E,), _f32),
        ),
        mesh=_MESH,
        scratch_types=[
            pltpu.VMEM((G,), jnp.int32),
            pltpu.VMEM((G,), jnp.int32),
            pltpu.VMEM((G,), jnp.int32),
            pltpu.VMEM((G, D), _f32),
            pltpu.VMEM((G, D), _f32),
            pltpu.VMEM((G, 128), _f32),
            pltpu.VMEM((G, W), _f32),
            pltpu.VMEM((16,), _f32),
            pltpu.VMEM((16,), _f32),
            pltpu.VMEM((G,), _f32),
            pltpu.VMEM((16,), _f32),
            pltpu.VMEM_SHARED((AROWS, W), _f32),
        ],
    )
    def kern(q_hbm, k_hbm, v_hbm, src_hbm, dst_hbm, pm_hbm, acc_hbm, gate_hbm,
             src_v, dst_v, srcl_v, qbuf, kbuf, vbuf, rowbuf, sstage, exstage,
             gbuf, pmv, acc):
        c = lax.axis_index("c")
        s = lax.axis_index("s")
        pltpu.sync_copy(pm_hbm, pmv)
        sstage[...] = jnp.zeros((16,), _f32)

        _zero_rows(rowbuf, G, W)

        @pl.loop(0, RPS // G)
        def _(i):
            pltpu.sync_copy(rowbuf, acc.at[pl.ds(s * RPS + i * G, G)])

        plsc.subcore_barrier()

        lanes = lax.iota(jnp.int32, 16)
        pmvec = pmv[...]

        @pl.loop(0, EPW // G)
        def _chunk(i):
            e0 = s * EPW + i * G
            pltpu.sync_copy(src_hbm.at[pl.ds(e0, G)], src_v)
            pltpu.sync_copy(dst_hbm.at[pl.ds(e0, G)], dst_v)
            for j in range(G // 16):
                srcl_v[pl.ds(j * 16, 16)] = src_v[pl.ds(j * 16, 16)] + c * N
            pltpu.sync_copy(q_hbm.at[dst_v], qbuf)
            pltpu.sync_copy(k_hbm.at[src_v], kbuf)
            pltpu.sync_copy(v_hbm.at[srcl_v], vbuf)

            @pl.loop(0, G)
            def _edge(e):
                for h in range(H):
                    t = (qbuf[e, pl.ds(h * 32, 16)] * kbuf[e, pl.ds(h * 32, 16)]
                         + qbuf[e, pl.ds(h * 32 + 16, 16)]
                         * kbuf[e, pl.ds(h * 32 + 16, 16)])
                    sstage[h] = jnp.sum(t)
                sv = sstage[...]
                ex = jnp.exp(sv * INV_SQRT_C)
                ex = jnp.where(lanes < 8, ex, 0.0)
                tg = jnp.sum(sv * pmvec)
                tgv = jnp.broadcast_to(tg, (16,)) * INV_SQRT_H
                bv = 1.0 / (1.0 + jnp.exp(-tgv))
                gv = jnp.where(bv >= 0.5, 1.0, 0.0)
                gbuf[e] = jnp.max(gv)
                rowbuf[e, pl.ds(0, 16)] = ex
                exstage[...] = ex
                for hl in range(4):
                    w = exstage[c * 4 + hl]
                    for j2 in range(2):
                        rowbuf[e, pl.ds(16 + hl * 32 + j2 * 16, 16)] = (
                            vbuf[e, pl.ds(hl * 32 + j2 * 16, 16)] * w)

            pltpu.sync_copy(rowbuf, acc.at[dst_v], add=True)

            @pl.when(c == 0)
            def _():
                pltpu.sync_copy(gbuf, gate_hbm.at[pl.ds(e0, G)])

        plsc.subcore_barrier()
        pltpu.sync_copy(acc.at[pl.ds(s * RPS, RPS)],
                        acc_hbm.at[pl.ds(c * AROWS + s * RPS, RPS)])

    return kern(q, k, vf, src, dst, pm16)


def _sc_layer(qf, kf, vf, src, dst, gate):
    @functools.partial(
        pl.kernel,
        out_type=jax.ShapeDtypeStruct((2 * AROWS, W), _f32),
        mesh=_MESH,
        scratch_types=[
            pltpu.VMEM((G,), jnp.int32),
            pltpu.VMEM((G,), jnp.int32),
            pltpu.VMEM((G,), jnp.int32),
            pltpu.VMEM((G,), jnp.int32),
            pltpu.VMEM((G, 128), _f32),
            pltpu.VMEM((G, 128), _f32),
            pltpu.VMEM((G, 128), _f32),
            pltpu.VMEM((G, W), _f32),
            pltpu.VMEM((16,), _f32),
            pltpu.VMEM((16,), _f32),
            pltpu.VMEM((G,), _f32),
            pltpu.VMEM_SHARED((AROWS, W), _f32),
        ],
    )
    def kern(q_hbm, k_hbm, v_hbm, src_hbm, dst_hbm, gate_hbm, acc_hbm,
             src_v, dst_v, srcl_v, dstl_v, qbuf, kbuf, vbuf, rowbuf, sstage,
             exstage, gbuf, acc):
        c = lax.axis_index("c")
        s = lax.axis_index("s")
        sstage[...] = jnp.zeros((16,), _f32)

        _zero_rows(rowbuf, G, W)

        @pl.loop(0, RPS // G)
        def _(i):
            pltpu.sync_copy(rowbuf, acc.at[pl.ds(s * RPS + i * G, G)])

        plsc.subcore_barrier()

        lanes = lax.iota(jnp.int32, 16)

        @pl.loop(0, EPW // G)
        def _chunk(i):
            e0 = s * EPW + i * G
            pltpu.sync_copy(src_hbm.at[pl.ds(e0, G)], src_v)
            pltpu.sync_copy(dst_hbm.at[pl.ds(e0, G)], dst_v)
            pltpu.sync_copy(gate_hbm.at[pl.ds(e0, G)], gbuf)
            for j in range(G // 16):
                srcl_v[pl.ds(j * 16, 16)] = src_v[pl.ds(j * 16, 16)] + c * N
                dstl_v[pl.ds(j * 16, 16)] = dst_v[pl.ds(j * 16, 16)] + c * N
            pltpu.sync_copy(q_hbm.at[dstl_v], qbuf)
            pltpu.sync_copy(k_hbm.at[srcl_v], kbuf)
            pltpu.sync_copy(v_hbm.at[srcl_v], vbuf)

            @pl.loop(0, G)
            def _edge(e):
                for hl in range(4):
                    t = (qbuf[e, pl.ds(hl * 32, 16)]
                         * kbuf[e, pl.ds(hl * 32, 16)]
                         + qbuf[e, pl.ds(hl * 32 + 16, 16)]
                         * kbuf[e, pl.ds(hl * 32 + 16, 16)])
                    sstage[hl] = jnp.sum(t)
                sv = sstage[...]
                ex = jnp.exp(sv * INV_SQRT_C)
                ex = jnp.where(lanes < 4, ex, 0.0)
                g = gbuf[e]
                ex = ex * g
                rowbuf[e, pl.ds(0, 16)] = ex
                exstage[...] = ex
                for hl in range(4):
                    w = exstage[hl]
                    for j2 in range(2):
                        rowbuf[e, pl.ds(16 + hl * 32 + j2 * 16, 16)] = (
                            vbuf[e, pl.ds(hl * 32 + j2 * 16, 16)] * w)

            pltpu.sync_copy(rowbuf, acc.at[dst_v], add=True)

        plsc.subcore_barrier()
        pltpu.sync_copy(acc.at[pl.ds(s * RPS, RPS)],
                        acc_hbm.at[pl.ds(c * AROWS + s * RPS, RPS)])

    return kern(qf, kf, vf, src, dst, gate)


_BN = 1000
_DOT = dict(preferred_element_type=jnp.float32,
            precision=jax.lax.Precision.HIGHEST)


def _head_expand():
    rows = lax.broadcasted_iota(jnp.int32, (H, D), 0)
    cols = lax.broadcasted_iota(jnp.int32, (H, D), 1)
    return jnp.where(rows == cols // C, 1.0, 0.0).astype(_f32)


def _gelu(x):
    return 0.5 * x * (1.0 + lax.erf(x * (1.0 / np.sqrt(2.0))))


def _layer_norm(x, g, b):
    mu = jnp.mean(x, axis=-1, keepdims=True)
    var = jnp.mean((x - mu) ** 2, axis=-1, keepdims=True)
    return (x - mu) / jnp.sqrt(var + 1e-5) * g + b


def _split_store(ref, full):
    ref[0, :, :] = full[:, :128]
    ref[1, :, :] = full[:, 128:]


def _tc_project0(x, wq, wk, wv):
    def body(x_ref, wq_ref, wk_ref, wv_ref, q_ref, k_ref, v_ref):
        xb = x_ref[...]
        q_ref[...] = jnp.dot(xb, wq_ref[...], **_DOT)
        k_ref[...] = jnp.dot(xb, wk_ref[...], **_DOT)
        _split_store(v_ref, jnp.dot(xb, wv_ref[...], **_DOT))

    wspec = pl.BlockSpec((D, D), lambda i: (0, 0))
    return pl.pallas_call(
        body,
        grid=(N // _BN,),
        in_specs=[pl.BlockSpec((_BN, D), lambda i: (i, 0)), wspec, wspec,
                  wspec],
        out_specs=[pl.BlockSpec((_BN, D), lambda i: (i, 0)),
                   pl.BlockSpec((_BN, D), lambda i: (i, 0)),
                   pl.BlockSpec((2, _BN, 128), lambda i: (0, i, 0))],
        out_shape=[jax.ShapeDtypeStruct((N, D), _f32),
                   jax.ShapeDtypeStruct((N, D), _f32),
                   jax.ShapeDtypeStruct((2, N, 128), _f32)],
    )(x, wq, wk, wv)


def _combine_acc(acc_ref, layer0):
    a0 = acc_ref[0]
    a1 = acc_ref[1]
    agg = jnp.concatenate([a0[:, 16:W], a1[:, 16:W]], axis=1)
    if layer0:
        den8 = a0[:, 0:8]
    else:
        den8 = jnp.concatenate([a0[:, 0:4], a1[:, 0:4]], axis=1)
    r8 = 1.0 / (den8 + 1e-16)
    rfull = jnp.dot(r8, _head_expand(), **_DOT)
    return agg * rfull


def _tc_mid(h, accf, ws, wq, wk, wv, g, b, layer0):
    acc = accf.reshape(2, AROWS, W)

    def body(h_ref, acc_ref, ws_ref, wq_ref, wk_ref, wv_ref, g_ref, b_ref,
             h_out, q_ref, k_ref, v_ref):
        hb = h_ref[...]
        o = _combine_acc(acc_ref, layer0) + jnp.dot(hb, ws_ref[...], **_DOT)
        o = _layer_norm(o, g_ref[...], b_ref[...])
        o = _gelu(o) + hb
        h_out[...] = o
        _split_store(q_ref, jnp.dot(o, wq_ref[...], **_DOT))
        _split_store(k_ref, jnp.dot(o, wk_ref[...], **_DOT))
        _split_store(v_ref, jnp.dot(o, wv_ref[...], **_DOT))

    wspec = pl.BlockSpec((D, D), lambda i: (0, 0))
    vspec = pl.BlockSpec((2, _BN, 128), lambda i: (0, i, 0))
    return pl.pallas_call(
        body,
        grid=(N // _BN,),
        in_specs=[pl.BlockSpec((_BN, D), lambda i: (i, 0)),
                  pl.BlockSpec((2, _BN, W), lambda i: (0, i, 0)),
                  wspec, wspec, wspec, wspec,
                  pl.BlockSpec((1, D), lambda i: (0, 0)),
                  pl.BlockSpec((1, D), lambda i: (0, 0))],
        out_specs=[pl.BlockSpec((_BN, D), lambda i: (i, 0)), vspec, vspec,
                   vspec],
        out_shape=[jax.ShapeDtypeStruct((N, D), _f32),
                   jax.ShapeDtypeStruct((2, N, 128), _f32),
                   jax.ShapeDtypeStruct((2, N, 128), _f32),
                   jax.ShapeDtypeStruct((2, N, 128), _f32)],
    )(h, acc, ws, wq, wk, wv, g, b)


def _tc_final(h, accf, ws):
    acc = accf.reshape(2, AROWS, W)

    def body(h_ref, acc_ref, ws_ref, out_ref):
        o = (_combine_acc(acc_ref, False)
             + jnp.dot(h_ref[...], ws_ref[...], **_DOT))
        out_ref[...] = _gelu(o)

    return pl.pallas_call(
        body,
        grid=(N // _BN,),
        in_specs=[pl.BlockSpec((_BN, D), lambda i: (i, 0)),
                  pl.BlockSpec((2, _BN, W), lambda i: (0, i, 0)),
                  pl.BlockSpec((D, D), lambda i: (0, 0))],
        out_specs=pl.BlockSpec((_BN, D), lambda i: (i, 0)),
        out_shape=jax.ShapeDtypeStruct((N, D), _f32),
    )(h, acc, ws)


def kernel(x, edge_index, Wq, Wk, Wv, Ws, param_multi, ln_g, ln_b):
    src = edge_index[0].astype(jnp.int32)
    dst = edge_index[1].astype(jnp.int32)
    pm16 = jnp.concatenate([param_multi.astype(_f32), jnp.zeros((8,), _f32)])

    q0, k0, v2 = _tc_project0(x, Wq[0], Wk[0], Wv[0])
    accf, gate = _sc_layer0(q0, k0, v2.reshape(2 * N, 128), src, dst, pm16)

    h = x
    for i in range(1, 5):
        h, q2, k2, v2 = _tc_mid(h, accf, Ws[i - 1], Wq[i], Wk[i], Wv[i],
                                ln_g[i - 1][None, :], ln_b[i - 1][None, :],
                                layer0=(i == 1))
        accf = _sc_layer(q2.reshape(2 * N, 128), k2.reshape(2 * N, 128),
                         v2.reshape(2 * N, 128), src, dst, gate)

    return _tc_final(h, accf, Ws[4])


# speedup vs baseline: 10.3394x; 10.3394x over previous
import dataclasses
import functools

import jax
import jax.numpy as jnp
import numpy as np
from jax import lax
from jax.experimental import pallas as pl
from jax.experimental.pallas import tpu as pltpu
from jax.experimental.pallas import tpu_sc as plsc

N = 10000
E = 160000
D = 256
H = 8
C = 32

NS = 16
AROWS = 10240
RPS = AROWS // NS
W = 80
G = 80
EPW = E // NS
INV_SQRT_C = float(1.0 / np.sqrt(C))
INV_SQRT_H = float(1.0 / np.sqrt(H))

_MESH = plsc.VectorSubcoreMesh(core_axis_name="c", subcore_axis_name="s")
_f32 = jnp.float32

_SC_PARAMS = pltpu.CompilerParams()
if "needs_layout_passes" in pltpu.CompilerParams.__dataclass_fields__:
    _SC_PARAMS = dataclasses.replace(_SC_PARAMS, needs_layout_passes=False)
if "use_tc_tiling_on_sc" in pltpu.CompilerParams.__dataclass_fields__:
    _SC_PARAMS = dataclasses.replace(_SC_PARAMS, use_tc_tiling_on_sc=False)


def _zero_rows(rowbuf, nrows, width):
    z = jnp.zeros((16,), _f32)

    @pl.loop(0, nrows)
    def _(r):
        for j in range(width // 16):
            rowbuf[r, pl.ds(j * 16, 16)] = z


def _zero_acc(acc, rowbuf, s):
    _zero_rows(rowbuf, G, W)

    @pl.loop(0, RPS // G)
    def _(i):
        pltpu.sync_copy(rowbuf, acc.at[pl.ds(s * RPS + i * G, G)])


def _store_weighted(rowbuf, vbuf, e, w0, w1):
    rowbuf[e, pl.ds(16, 16)] = vbuf[e, pl.ds(0, 16)] * w0
    rowbuf[e, pl.ds(32, 16)] = vbuf[e, pl.ds(16, 16)] * w0
    rowbuf[e, pl.ds(48, 16)] = vbuf[e, pl.ds(32, 16)] * w1
    rowbuf[e, pl.ds(64, 16)] = vbuf[e, pl.ds(48, 16)] * w1


def _sc_layer0(q, k, vf, src, dst, pm16):
    @functools.partial(
        pl.kernel,
        out_type=(
            jax.ShapeDtypeStruct((4 * AROWS, W), _f32),
            jax.ShapeDtypeStruct((E,), _f32),
            jax.ShapeDtypeStruct((E, 16), _f32),
        ),
        mesh=_MESH,
        compiler_params=_SC_PARAMS,
        scratch_types=[
            pltpu.VMEM((G,), jnp.int32),
            pltpu.VMEM((G,), jnp.int32),
            pltpu.VMEM((G,), jnp.int32),
            pltpu.VMEM((G, D), _f32),
            pltpu.VMEM((G, D), _f32),
            pltpu.VMEM((G, 64), _f32),
            pltpu.VMEM((G, W), _f32),
            pltpu.VMEM((G, 16), _f32),
            pltpu.VMEM((G,), _f32),
            pltpu.VMEM((16,), _f32),
            pltpu.VMEM_SHARED((AROWS, W), _f32),
        ],
    )
    def kern(q_hbm, k_hbm, v_hbm, src_hbm, dst_hbm, pm_hbm, acc_hbm, gate_hbm,
             sc_hbm, src_v, dst_v, srcl_v, qbuf, kbuf, vbuf, rowbuf, svbuf,
             gbuf, pmv, acc):
        c = lax.axis_index("c")
        s = lax.axis_index("s")
        pltpu.sync_copy(pm_hbm, pmv)

        _zero_acc(acc, rowbuf, s)
        plsc.subcore_barrier()

        lanes = lax.iota(jnp.int32, 16)
        pmvec = pmv[...]

        @pl.loop(0, EPW // G)
        def _chunk(i):
            e0 = s * EPW + i * G
            pltpu.sync_copy(src_hbm.at[pl.ds(e0, G)], src_v)
            pltpu.sync_copy(dst_hbm.at[pl.ds(e0, G)], dst_v)
            for j in range(G // 16):
                srcl_v[pl.ds(j * 16, 16)] = (src_v[pl.ds(j * 16, 16)]
                                             + (c * 2) * N)
            pltpu.sync_copy(q_hbm.at[dst_v], qbuf)
            pltpu.sync_copy(k_hbm.at[src_v], kbuf)
            pltpu.sync_copy(v_hbm.at[srcl_v], vbuf)

            @pl.loop(0, G)
            def _edge(e):
                sv = jnp.zeros((16,), _f32)
                for h in range(H):
                    t = (qbuf[e, pl.ds(h * 32, 16)] * kbuf[e, pl.ds(h * 32, 16)]
                         + qbuf[e, pl.ds(h * 32 + 16, 16)]
                         * kbuf[e, pl.ds(h * 32 + 16, 16)])
                    sv = jnp.where(lanes == h, jnp.sum(t), sv)
                svbuf[e, pl.ds(0, 16)] = sv
                ex = jnp.exp(sv * INV_SQRT_C)
                w0 = jnp.where(c == 0, ex[0], ex[4])
                w1 = jnp.where(c == 0, ex[1], ex[5])
                exrow = jnp.where(lanes == 0, w0, 0.0)
                exrow = jnp.where(lanes == 1, w1, exrow)
                rowbuf[e, pl.ds(0, 16)] = exrow
                _store_weighted(rowbuf, vbuf, e, w0, w1)

            pltpu.sync_copy(rowbuf, acc.at[dst_v], add=True)
            pltpu.sync_copy(svbuf, sc_hbm.at[pl.ds(e0, G)])

            for j in range(G // 16):
                tg = jnp.zeros((16,), _f32)
                for h in range(H):
                    col = plsc.load_gather(
                        svbuf, [lanes + j * 16, jnp.full((16,), h, jnp.int32)])
                    tg = tg + col * pmvec[h]
                bv = 1.0 / (1.0 + jnp.exp(-tg * INV_SQRT_H))
                gbuf[pl.ds(j * 16, 16)] = jnp.where(bv >= 0.5, 1.0, 0.0)

            @pl.when(c == 0)
            def _():
                pltpu.sync_copy(gbuf, gate_hbm.at[pl.ds(e0, G)])

        plsc.subcore_barrier()
        pltpu.sync_copy(acc.at[pl.ds(s * RPS, RPS)],
                        acc_hbm.at[pl.ds((c * 2) * AROWS + s * RPS, RPS)])
        plsc.subcore_barrier()

        _zero_acc(acc, rowbuf, s)
        plsc.subcore_barrier()

        @pl.loop(0, EPW // G)
        def _chunk1(i):
            e0 = s * EPW + i * G
            pltpu.sync_copy(src_hbm.at[pl.ds(e0, G)], src_v)
            pltpu.sync_copy(dst_hbm.at[pl.ds(e0, G)], dst_v)
            for j in range(G // 16):
                srcl_v[pl.ds(j * 16, 16)] = (src_v[pl.ds(j * 16, 16)]
                                             + (c * 2 + 1) * N)
            pltpu.sync_copy(v_hbm.at[srcl_v], vbuf)
            pltpu.sync_copy(sc_hbm.at[pl.ds(e0, G)], svbuf)

            @pl.loop(0, G)
            def _edge(e):
                sv = svbuf[e, pl.ds(0, 16)]
                ex = jnp.exp(sv * INV_SQRT_C)
                w0 = jnp.where(c == 0, ex[2], ex[6])
                w1 = jnp.where(c == 0, ex[3], ex[7])
                exrow = jnp.where(lanes == 0, w0, 0.0)
                exrow = jnp.where(lanes == 1, w1, exrow)
                rowbuf[e, pl.ds(0, 16)] = exrow
                _store_weighted(rowbuf, vbuf, e, w0, w1)

            pltpu.sync_copy(rowbuf, acc.at[dst_v], add=True)

        plsc.subcore_barrier()
        pltpu.sync_copy(acc.at[pl.ds(s * RPS, RPS)],
                        acc_hbm.at[pl.ds((c * 2 + 1) * AROWS + s * RPS, RPS)])

    return kern(q, k, vf, src, dst, pm16)


def _sc_layer(qf, kf, vf, src, dst, gate):
    @functools.partial(
        pl.kernel,
        out_type=jax.ShapeDtypeStruct((4 * AROWS, W), _f32),
        mesh=_MESH,
        compiler_params=_SC_PARAMS,
        scratch_types=[
            pltpu.VMEM((G,), jnp.int32),
            pltpu.VMEM((G,), jnp.int32),
            pltpu.VMEM((G,), jnp.int32),
            pltpu.VMEM((G,), jnp.int32),
            pltpu.VMEM((G, 64), _f32),
            pltpu.VMEM((G, 64), _f32),
            pltpu.VMEM((G, 64), _f32),
            pltpu.VMEM((G, W), _f32),
            pltpu.VMEM((G,), _f32),
            pltpu.VMEM_SHARED((AROWS, W), _f32),
        ],
    )
    def kern(q_hbm, k_hbm, v_hbm, src_hbm, dst_hbm, gate_hbm, acc_hbm,
             src_v, dst_v, srcl_v, dstl_v, qbuf, kbuf, vbuf, rowbuf,
             gbuf, acc):
        c = lax.axis_index("c")
        s = lax.axis_index("s")

        lanes = lax.iota(jnp.int32, 16)

        for p in range(2):
            _zero_acc(acc, rowbuf, s)
            plsc.subcore_barrier()
            hp_off = (c * 2 + p) * N

            @pl.loop(0, EPW // G)
            def _chunk(i):
                e0 = s * EPW + i * G
                pltpu.sync_copy(src_hbm.at[pl.ds(e0, G)], src_v)
                pltpu.sync_copy(dst_hbm.at[pl.ds(e0, G)], dst_v)
                pltpu.sync_copy(gate_hbm.at[pl.ds(e0, G)], gbuf)
                for j in range(G // 16):
                    srcl_v[pl.ds(j * 16, 16)] = (src_v[pl.ds(j * 16, 16)]
                                                 + hp_off)
                    dstl_v[pl.ds(j * 16, 16)] = (dst_v[pl.ds(j * 16, 16)]
                                                 + hp_off)
                pltpu.sync_copy(q_hbm.at[dstl_v], qbuf)
                pltpu.sync_copy(k_hbm.at[srcl_v], kbuf)
                pltpu.sync_copy(v_hbm.at[srcl_v], vbuf)

                @pl.loop(0, G)
                def _edge(e):
                    sv = jnp.zeros((16,), _f32)
                    for hl in range(2):
                        t = (qbuf[e, pl.ds(hl * 32, 16)]
                             * kbuf[e, pl.ds(hl * 32, 16)]
                             + qbuf[e, pl.ds(hl * 32 + 16, 16)]
                             * kbuf[e, pl.ds(hl * 32 + 16, 16)])
                        sv = jnp.where(lanes == hl, jnp.sum(t), sv)
                    ex = jnp.exp(sv * INV_SQRT_C)
                    ex = jnp.where(lanes < 2, ex, 0.0)
                    gvec = plsc.load_gather(
                        gbuf, [jnp.full((16,), 1, jnp.int32) * e])
                    ex = ex * gvec
                    rowbuf[e, pl.ds(0, 16)] = ex
                    _store_weighted(rowbuf, vbuf, e, ex[0], ex[1])

                pltpu.sync_copy(rowbuf, acc.at[dst_v], add=True)

            plsc.subcore_barrier()
            pltpu.sync_copy(
                acc.at[pl.ds(s * RPS, RPS)],
                acc_hbm.at[pl.ds((c * 2 + p) * AROWS + s * RPS, RPS)])
            if p == 0:
                plsc.subcore_barrier()

    return kern(qf, kf, vf, src, dst, gate)


_BN = 1000
_DOT = dict(preferred_element_type=jnp.float32,
            precision=jax.lax.Precision.DEFAULT)
_DOT_HI = dict(preferred_element_type=jnp.float32,
               precision=jax.lax.Precision.HIGHEST)


def _head_expand():
    rows = lax.broadcasted_iota(jnp.int32, (H, D), 0)
    cols = lax.broadcasted_iota(jnp.int32, (H, D), 1)
    return jnp.where(rows == cols // C, 1.0, 0.0).astype(_f32)


def _gelu(x):
    return 0.5 * x * (1.0 + lax.erf(x * (1.0 / np.sqrt(2.0))))


def _layer_norm(x, g, b):
    mu = jnp.mean(x, axis=-1, keepdims=True)
    var = jnp.mean((x - mu) ** 2, axis=-1, keepdims=True)
    return (x - mu) / jnp.sqrt(var + 1e-5) * g + b


def _split_store4(ref, full):
    for hp in range(4):
        ref[hp, :, :] = full[:, hp * 64:(hp + 1) * 64]


def _tc_project0(x, wq, wk, wv):
    def body(x_ref, wq_ref, wk_ref, wv_ref, q_ref, k_ref, v_ref):
        xb = x_ref[...]
        q_ref[...] = jnp.dot(xb, wq_ref[...], **_DOT)
        k_ref[...] = jnp.dot(xb, wk_ref[...], **_DOT)
        _split_store4(v_ref, jnp.dot(xb, wv_ref[...], **_DOT))

    wspec = pl.BlockSpec((D, D), lambda i: (0, 0))
    return pl.pallas_call(
        body,
        grid=(N // _BN,),
        in_specs=[pl.BlockSpec((_BN, D), lambda i: (i, 0)), wspec, wspec,
                  wspec],
        out_specs=[pl.BlockSpec((_BN, D), lambda i: (i, 0)),
                   pl.BlockSpec((_BN, D), lambda i: (i, 0)),
                   pl.BlockSpec((4, _BN, 64), lambda i: (0, i, 0))],
        out_shape=[jax.ShapeDtypeStruct((N, D), _f32),
                   jax.ShapeDtypeStruct((N, D), _f32),
                   jax.ShapeDtypeStruct((4, N, 64), _f32)],
    )(x, wq, wk, wv)


def _combine_acc(acc_ref):
    agg = jnp.concatenate([acc_ref[hp][:, 16:W] for hp in range(4)], axis=1)
    den8 = jnp.concatenate([acc_ref[hp][:, 0:2] for hp in range(4)], axis=1)
    r8 = 1.0 / (den8 + 1e-16)
    rfull = jnp.dot(r8, _head_expand(), **_DOT_HI)
    return agg * rfull


def _tc_mid(h, accf, ws, wq, wk, wv, g, b):
    acc = accf.reshape(4, AROWS, W)

    def body(h_ref, acc_ref, ws_ref, wq_ref, wk_ref, wv_ref, g_ref, b_ref,
             h_out, q_ref, k_ref, v_ref):
        hb = h_ref[...]
        o = _combine_acc(acc_ref) + jnp.dot(hb, ws_ref[...], **_DOT)
        o = _layer_norm(o, g_ref[...], b_ref[...])
        o = _gelu(o) + hb
        h_out[...] = o
        _split_store4(q_ref, jnp.dot(o, wq_ref[...], **_DOT))
        _split_store4(k_ref, jnp.dot(o, wk_ref[...], **_DOT))
        _split_store4(v_ref, jnp.dot(o, wv_ref[...], **_DOT))

    wspec = pl.BlockSpec((D, D), lambda i: (0, 0))
    vspec = pl.BlockSpec((4, _BN, 64), lambda i: (0, i, 0))
    return pl.pallas_call(
        body,
        grid=(N // _BN,),
        in_specs=[pl.BlockSpec((_BN, D), lambda i: (i, 0)),
                  pl.BlockSpec((4, _BN, W), lambda i: (0, i, 0)),
                  wspec, wspec, wspec, wspec,
                  pl.BlockSpec((1, D), lambda i: (0, 0)),
                  pl.BlockSpec((1, D), lambda i: (0, 0))],
        out_specs=[pl.BlockSpec((_BN, D), lambda i: (i, 0)), vspec, vspec,
                   vspec],
        out_shape=[jax.ShapeDtypeStruct((N, D), _f32),
                   jax.ShapeDtypeStruct((4, N, 64), _f32),
                   jax.ShapeDtypeStruct((4, N, 64), _f32),
                   jax.ShapeDtypeStruct((4, N, 64), _f32)],
    )(h, acc, ws, wq, wk, wv, g, b)


def _tc_final(h, accf, ws):
    acc = accf.reshape(4, AROWS, W)

    def body(h_ref, acc_ref, ws_ref, out_ref):
        o = _combine_acc(acc_ref) + jnp.dot(h_ref[...], ws_ref[...], **_DOT)
        out_ref[...] = _gelu(o)

    return pl.pallas_call(
        body,
        grid=(N // _BN,),
        in_specs=[pl.BlockSpec((_BN, D), lambda i: (i, 0)),
                  pl.BlockSpec((4, _BN, W), lambda i: (0, i, 0)),
                  pl.BlockSpec((D, D), lambda i: (0, 0))],
        out_specs=pl.BlockSpec((_BN, D), lambda i: (i, 0)),
        out_shape=jax.ShapeDtypeStruct((N, D), _f32),
    )(h, acc, ws)


def kernel(x, edge_index, Wq, Wk, Wv, Ws, param_multi, ln_g, ln_b):
    src = edge_index[0].astype(jnp.int32)
    dst = edge_index[1].astype(jnp.int32)
    pm16 = jnp.concatenate([param_multi.astype(_f32), jnp.zeros((8,), _f32)])

    q0, k0, v4 = _tc_project0(x, Wq[0], Wk[0], Wv[0])
    accf, gate, _ = _sc_layer0(q0, k0, v4.reshape(4 * N, 64), src, dst, pm16)

    h = x
    for i in range(1, 5):
        h, q4, k4, v4 = _tc_mid(h, accf, Ws[i - 1], Wq[i], Wk[i], Wv[i],
                                ln_g[i - 1][None, :], ln_b[i - 1][None, :])
        accf = _sc_layer(q4.reshape(4 * N, 64), k4.reshape(4 * N, 64),
                         v4.reshape(4 * N, 64), src, dst, gate)

    return _tc_final(h, accf, Ws[4])


# speedup vs baseline: 15.5253x; 1.5016x over previous
import dataclasses
import functools

import jax
import jax.numpy as jnp
import numpy as np
from jax import lax
from jax.experimental import pallas as pl
from jax.experimental.pallas import tpu as pltpu
from jax.experimental.pallas import tpu_sc as plsc

N = 10000
E = 160000
D = 256
H = 8
C = 32

NS = 16
AROWS = 10240
RPS = AROWS // NS
W = 80
G = 80
EPW = E // NS
INV_SQRT_C = float(1.0 / np.sqrt(C))
INV_SQRT_H = float(1.0 / np.sqrt(H))

_MESH = plsc.VectorSubcoreMesh(core_axis_name="c", subcore_axis_name="s")
_f32 = jnp.float32

_SC_PARAMS = pltpu.CompilerParams()
if "needs_layout_passes" in pltpu.CompilerParams.__dataclass_fields__:
    _SC_PARAMS = dataclasses.replace(_SC_PARAMS, needs_layout_passes=False)
if "use_tc_tiling_on_sc" in pltpu.CompilerParams.__dataclass_fields__:
    _SC_PARAMS = dataclasses.replace(_SC_PARAMS, use_tc_tiling_on_sc=False)


def _zero_rows(rowbuf, nrows, width):
    z = jnp.zeros((16,), _f32)

    @pl.loop(0, nrows)
    def _(r):
        for j in range(width // 16):
            rowbuf[r, pl.ds(j * 16, 16)] = z


def _zero_acc(acc, rowbuf, s):
    _zero_rows(rowbuf, G, W)

    @pl.loop(0, RPS // G)
    def _(i):
        pltpu.sync_copy(rowbuf, acc.at[pl.ds(s * RPS + i * G, G)])


def _store_weighted(rowbuf, vbuf, e, w0, w1):
    rowbuf[e, pl.ds(16, 16)] = vbuf[e, pl.ds(0, 16)] * w0
    rowbuf[e, pl.ds(32, 16)] = vbuf[e, pl.ds(16, 16)] * w0
    rowbuf[e, pl.ds(48, 16)] = vbuf[e, pl.ds(32, 16)] * w1
    rowbuf[e, pl.ds(64, 16)] = vbuf[e, pl.ds(48, 16)] * w1


def _sc_layer0(q, k, vf, src, dst, pm16):
    @functools.partial(
        pl.kernel,
        out_type=(
            jax.ShapeDtypeStruct((4 * AROWS, W), _f32),
            jax.ShapeDtypeStruct((E,), _f32),
            jax.ShapeDtypeStruct((E, 16), _f32),
        ),
        mesh=_MESH,
        compiler_params=_SC_PARAMS,
        scratch_types=[
            pltpu.VMEM((G,), jnp.int32),
            pltpu.VMEM((G,), jnp.int32),
            pltpu.VMEM((G,), jnp.int32),
            pltpu.VMEM((G, D), _f32),
            pltpu.VMEM((G, D), _f32),
            pltpu.VMEM((G, 64), _f32),
            pltpu.VMEM((G, W), _f32),
            pltpu.VMEM((G, 16), _f32),
            pltpu.VMEM((G,), _f32),
            pltpu.VMEM((16,), _f32),
            pltpu.VMEM_SHARED((AROWS, W), _f32),
        ],
    )
    def kern(q_hbm, k_hbm, v_hbm, src_hbm, dst_hbm, pm_hbm, acc_hbm, gate_hbm,
             sc_hbm, src_v, dst_v, srcl_v, qbuf, kbuf, vbuf, rowbuf, svbuf,
             gbuf, pmv, acc):
        c = lax.axis_index("c")
        s = lax.axis_index("s")
        pltpu.sync_copy(pm_hbm, pmv)

        _zero_acc(acc, rowbuf, s)
        plsc.subcore_barrier()

        lanes = lax.iota(jnp.int32, 16)
        pmvec = pmv[...]

        @pl.loop(0, EPW // G)
        def _chunk(i):
            e0 = s * EPW + i * G
            pltpu.sync_copy(src_hbm.at[pl.ds(e0, G)], src_v)
            pltpu.sync_copy(dst_hbm.at[pl.ds(e0, G)], dst_v)
            for j in range(G // 16):
                srcl_v[pl.ds(j * 16, 16)] = (src_v[pl.ds(j * 16, 16)]
                                             + (c * 2) * N)
            pltpu.sync_copy(q_hbm.at[dst_v], qbuf)
            pltpu.sync_copy(k_hbm.at[src_v], kbuf)
            pltpu.sync_copy(v_hbm.at[srcl_v], vbuf)

            @pl.loop(0, G)
            def _edge(e):
                sv = jnp.zeros((16,), _f32)
                for h in range(H):
                    t = (qbuf[e, pl.ds(h * 32, 16)] * kbuf[e, pl.ds(h * 32, 16)]
                         + qbuf[e, pl.ds(h * 32 + 16, 16)]
                         * kbuf[e, pl.ds(h * 32 + 16, 16)])
                    sv = jnp.where(lanes == h, jnp.sum(t), sv)
                svbuf[e, pl.ds(0, 16)] = sv
                ex = jnp.exp(sv * INV_SQRT_C)
                w0 = jnp.where(c == 0, ex[0], ex[4])
                w1 = jnp.where(c == 0, ex[1], ex[5])
                exrow = jnp.where(lanes == 0, w0, 0.0)
                exrow = jnp.where(lanes == 1, w1, exrow)
                rowbuf[e, pl.ds(0, 16)] = exrow
                _store_weighted(rowbuf, vbuf, e, w0, w1)

            pltpu.sync_copy(rowbuf, acc.at[dst_v], add=True)
            pltpu.sync_copy(svbuf, sc_hbm.at[pl.ds(e0, G)])

            for j in range(G // 16):
                tg = jnp.zeros((16,), _f32)
                for h in range(H):
                    col = plsc.load_gather(
                        svbuf, [lanes + j * 16, jnp.full((16,), h, jnp.int32)])
                    tg = tg + col * pmvec[h]
                bv = 1.0 / (1.0 + jnp.exp(-tg * INV_SQRT_H))
                gbuf[pl.ds(j * 16, 16)] = jnp.where(bv >= 0.5, 1.0, 0.0)

            @pl.when(c == 0)
            def _():
                pltpu.sync_copy(gbuf, gate_hbm.at[pl.ds(e0, G)])

        plsc.subcore_barrier()
        pltpu.sync_copy(acc.at[pl.ds(s * RPS, RPS)],
                        acc_hbm.at[pl.ds((c * 2) * AROWS + s * RPS, RPS)])
        plsc.subcore_barrier()

        _zero_acc(acc, rowbuf, s)
        plsc.subcore_barrier()

        @pl.loop(0, EPW // G)
        def _chunk1(i):
            e0 = s * EPW + i * G
            pltpu.sync_copy(src_hbm.at[pl.ds(e0, G)], src_v)
            pltpu.sync_copy(dst_hbm.at[pl.ds(e0, G)], dst_v)
            for j in range(G // 16):
                srcl_v[pl.ds(j * 16, 16)] = (src_v[pl.ds(j * 16, 16)]
                                             + (c * 2 + 1) * N)
            pltpu.sync_copy(v_hbm.at[srcl_v], vbuf)
            pltpu.sync_copy(sc_hbm.at[pl.ds(e0, G)], svbuf)

            @pl.loop(0, G)
            def _edge(e):
                sv = svbuf[e, pl.ds(0, 16)]
                ex = jnp.exp(sv * INV_SQRT_C)
                w0 = jnp.where(c == 0, ex[2], ex[6])
                w1 = jnp.where(c == 0, ex[3], ex[7])
                exrow = jnp.where(lanes == 0, w0, 0.0)
                exrow = jnp.where(lanes == 1, w1, exrow)
                rowbuf[e, pl.ds(0, 16)] = exrow
                _store_weighted(rowbuf, vbuf, e, w0, w1)

            pltpu.sync_copy(rowbuf, acc.at[dst_v], add=True)

        plsc.subcore_barrier()
        pltpu.sync_copy(acc.at[pl.ds(s * RPS, RPS)],
                        acc_hbm.at[pl.ds((c * 2 + 1) * AROWS + s * RPS, RPS)])

    return kern(q, k, vf, src, dst, pm16)


NCH = EPW // G


def _sc_layer(qf, kf, vf, src, dst, gate):
    @functools.partial(
        pl.kernel,
        out_type=jax.ShapeDtypeStruct((4 * AROWS, W), _f32),
        mesh=_MESH,
        compiler_params=_SC_PARAMS,
        scratch_types=[
            pltpu.VMEM((2, G), jnp.int32),
            pltpu.VMEM((2, G), jnp.int32),
            pltpu.VMEM((2, G), jnp.int32),
            pltpu.VMEM((2, G), jnp.int32),
            pltpu.VMEM((2, G), jnp.int32),
            pltpu.VMEM((2 * G, 64), _f32),
            pltpu.VMEM((2 * G, 64), _f32),
            pltpu.VMEM((2 * G, 64), _f32),
            pltpu.VMEM((2 * G, W), _f32),
            pltpu.VMEM((2, G), _f32),
            pltpu.VMEM_SHARED((AROWS, W), _f32),
            pltpu.SemaphoreType.DMA,
            pltpu.SemaphoreType.DMA,
            pltpu.SemaphoreType.DMA,
            pltpu.SemaphoreType.DMA,
            pltpu.SemaphoreType.DMA,
            pltpu.SemaphoreType.DMA,
        ],
    )
    def kern(q_hbm, k_hbm, v_hbm, src_hbm, dst_hbm, gate_hbm, acc_hbm,
             src_v, dst_v, srcl_v, dstl_v, dsts_v, qbuf, kbuf, vbuf, rowbuf,
             gbuf, acc, sem_i0, sem_i1, sem_g0, sem_g1, sem_s0, sem_s1):
        c = lax.axis_index("c")
        s = lax.axis_index("s")
        sem_i = [sem_i0, sem_i1]
        sem_g = [sem_g0, sem_g1]
        sem_s = [sem_s0, sem_s1]

        lanes = lax.iota(jnp.int32, 16)

        for p in range(2):
            _zero_acc(acc, rowbuf.at[pl.ds(0, G)], s)
            plsc.subcore_barrier()
            hp_off = (c * 2 + p) * N

            def issue_idx(i, b):
                e0 = s * EPW + i * G
                pltpu.async_copy(src_hbm.at[pl.ds(e0, G)], src_v.at[b],
                                 sem_i[b])
                pltpu.async_copy(dst_hbm.at[pl.ds(e0, G)], dst_v.at[b],
                                 sem_i[b])
                pltpu.async_copy(gate_hbm.at[pl.ds(e0, G)], gbuf.at[b],
                                 sem_i[b])

            def wait_idx(b):
                pltpu.make_async_copy(src_hbm.at[pl.ds(0, G)], src_v.at[b],
                                      sem_i[b]).wait()
                pltpu.make_async_copy(dst_hbm.at[pl.ds(0, G)], dst_v.at[b],
                                      sem_i[b]).wait()
                pltpu.make_async_copy(gate_hbm.at[pl.ds(0, G)], gbuf.at[b],
                                      sem_i[b]).wait()

            def issue_gathers(b):
                for j in range(G // 16):
                    srcl_v[b, pl.ds(j * 16, 16)] = (
                        src_v[b, pl.ds(j * 16, 16)] + hp_off)
                    dstl_v[b, pl.ds(j * 16, 16)] = (
                        dst_v[b, pl.ds(j * 16, 16)] + hp_off)
                pltpu.async_copy(q_hbm.at[dstl_v.at[b]],
                                 qbuf.at[pl.ds(b * G, G)], sem_g[b])
                pltpu.async_copy(k_hbm.at[srcl_v.at[b]],
                                 kbuf.at[pl.ds(b * G, G)], sem_g[b])
                pltpu.async_copy(v_hbm.at[srcl_v.at[b]],
                                 vbuf.at[pl.ds(b * G, G)], sem_g[b])

            def wait_gathers(b):
                pltpu.make_async_copy(q_hbm.at[dstl_v.at[b]],
                                      qbuf.at[pl.ds(b * G, G)],
                                      sem_g[b]).wait()
                pltpu.make_async_copy(k_hbm.at[srcl_v.at[b]],
                                      kbuf.at[pl.ds(b * G, G)],
                                      sem_g[b]).wait()
                pltpu.make_async_copy(v_hbm.at[srcl_v.at[b]],
                                      vbuf.at[pl.ds(b * G, G)],
                                      sem_g[b]).wait()

            def wait_scatter(b):
                pltpu.make_async_copy(rowbuf.at[pl.ds(b * G, G)],
                                      acc.at[dsts_v.at[b]], sem_s[b]).wait()

            def compute(b):
                for j in range(G // 16):
                    dsts_v[b, pl.ds(j * 16, 16)] = dst_v[b, pl.ds(j * 16, 16)]

                @pl.loop(0, G)
                def _edge(e):
                    r = b * G + e
                    sv = jnp.zeros((16,), _f32)
                    for hl in range(2):
                        t = (qbuf[r, pl.ds(hl * 32, 16)]
                             * kbuf[r, pl.ds(hl * 32, 16)]
                             + qbuf[r, pl.ds(hl * 32 + 16, 16)]
                             * kbuf[r, pl.ds(hl * 32 + 16, 16)])
                        sv = jnp.where(lanes == hl, jnp.sum(t), sv)
                    ex = jnp.exp(sv * INV_SQRT_C)
                    ex = jnp.where(lanes < 2, ex, 0.0)
                    gvec = plsc.load_gather(
                        gbuf, [jnp.full((16,), b, jnp.int32),
                               jnp.full((16,), 1, jnp.int32) * e])
                    ex = ex * gvec
                    rowbuf[r, pl.ds(0, 16)] = ex
                    _store_weighted(rowbuf, vbuf, r, ex[0], ex[1])

                pltpu.async_copy(rowbuf.at[pl.ds(b * G, G)],
                                 acc.at[dsts_v.at[b]], sem_s[b], add=True)

            issue_idx(0, 0)
            wait_idx(0)
            issue_gathers(0)
            issue_idx(1, 1)

            @pl.loop(0, NCH // 2)
            def _pair(ii):
                for b in range(2):
                    i = ii * 2 + b
                    wait_gathers(b)

                    @pl.when(i <= NCH - 2)
                    def _():
                        wait_idx(1 - b)
                        issue_gathers(1 - b)

                    @pl.when(i >= 2)
                    def _():
                        wait_scatter(b)

                    compute(b)

                    @pl.when(i <= NCH - 3)
                    def _():
                        issue_idx(i + 2, b)

            if NCH % 2 == 1:
                wait_gathers(0)
                wait_scatter(0)
                compute(0)
                wait_scatter(1)
                wait_scatter(0)
            else:
                wait_scatter(0)
                wait_scatter(1)

            plsc.subcore_barrier()
            pltpu.sync_copy(
                acc.at[pl.ds(s * RPS, RPS)],
                acc_hbm.at[pl.ds((c * 2 + p) * AROWS + s * RPS, RPS)])
            if p == 0:
                plsc.subcore_barrier()

    return kern(qf, kf, vf, src, dst, gate)


_BN = 1000
_DOT = dict(preferred_element_type=jnp.float32,
            precision=jax.lax.Precision.DEFAULT)
_DOT_HI = dict(preferred_element_type=jnp.float32,
               precision=jax.lax.Precision.HIGHEST)


def _head_expand():
    rows = lax.broadcasted_iota(jnp.int32, (H, D), 0)
    cols = lax.broadcasted_iota(jnp.int32, (H, D), 1)
    return jnp.where(rows == cols // C, 1.0, 0.0).astype(_f32)


def _gelu(x):
    return 0.5 * x * (1.0 + lax.erf(x * (1.0 / np.sqrt(2.0))))


def _layer_norm(x, g, b):
    mu = jnp.mean(x, axis=-1, keepdims=True)
    var = jnp.mean((x - mu) ** 2, axis=-1, keepdims=True)
    return (x - mu) / jnp.sqrt(var + 1e-5) * g + b


def _split_store4(ref, full):
    for hp in range(4):
        ref[hp, :, :] = full[:, hp * 64:(hp + 1) * 64]


def _tc_project0(x, wq, wk, wv):
    def body(x_ref, wq_ref, wk_ref, wv_ref, q_ref, k_ref, v_ref):
        xb = x_ref[...]
        q_ref[...] = jnp.dot(xb, wq_ref[...], **_DOT)
        k_ref[...] = jnp.dot(xb, wk_ref[...], **_DOT)
        _split_store4(v_ref, jnp.dot(xb, wv_ref[...], **_DOT))

    wspec = pl.BlockSpec((D, D), lambda i: (0, 0))
    return pl.pallas_call(
        body,
        grid=(N // _BN,),
        in_specs=[pl.BlockSpec((_BN, D), lambda i: (i, 0)), wspec, wspec,
                  wspec],
        out_specs=[pl.BlockSpec((_BN, D), lambda i: (i, 0)),
                   pl.BlockSpec((_BN, D), lambda i: (i, 0)),
                   pl.BlockSpec((4, _BN, 64), lambda i: (0, i, 0))],
        out_shape=[jax.ShapeDtypeStruct((N, D), _f32),
                   jax.ShapeDtypeStruct((N, D), _f32),
                   jax.ShapeDtypeStruct((4, N, 64), _f32)],
    )(x, wq, wk, wv)


def _combine_acc(acc_ref):
    agg = jnp.concatenate([acc_ref[hp][:, 16:W] for hp in range(4)], axis=1)
    den8 = jnp.concatenate([acc_ref[hp][:, 0:2] for hp in range(4)], axis=1)
    r8 = 1.0 / (den8 + 1e-16)
    rfull = jnp.dot(r8, _head_expand(), **_DOT_HI)
    return agg * rfull


def _tc_mid(h, accf, ws, wq, wk, wv, g, b):
    acc = accf.reshape(4, AROWS, W)

    def body(h_ref, acc_ref, ws_ref, wq_ref, wk_ref, wv_ref, g_ref, b_ref,
             h_out, q_ref, k_ref, v_ref):
        hb = h_ref[...]
        o = _combine_acc(acc_ref) + jnp.dot(hb, ws_ref[...], **_DOT)
        o = _layer_norm(o, g_ref[...], b_ref[...])
        o = _gelu(o) + hb
        h_out[...] = o
        _split_store4(q_ref, jnp.dot(o, wq_ref[...], **_DOT))
        _split_store4(k_ref, jnp.dot(o, wk_ref[...], **_DOT))
        _split_store4(v_ref, jnp.dot(o, wv_ref[...], **_DOT))

    wspec = pl.BlockSpec((D, D), lambda i: (0, 0))
    vspec = pl.BlockSpec((4, _BN, 64), lambda i: (0, i, 0))
    return pl.pallas_call(
        body,
        grid=(N // _BN,),
        in_specs=[pl.BlockSpec((_BN, D), lambda i: (i, 0)),
                  pl.BlockSpec((4, _BN, W), lambda i: (0, i, 0)),
                  wspec, wspec, wspec, wspec,
                  pl.BlockSpec((1, D), lambda i: (0, 0)),
                  pl.BlockSpec((1, D), lambda i: (0, 0))],
        out_specs=[pl.BlockSpec((_BN, D), lambda i: (i, 0)), vspec, vspec,
                   vspec],
        out_shape=[jax.ShapeDtypeStruct((N, D), _f32),
                   jax.ShapeDtypeStruct((4, N, 64), _f32),
                   jax.ShapeDtypeStruct((4, N, 64), _f32),
                   jax.ShapeDtypeStruct((4, N, 64), _f32)],
    )(h, acc, ws, wq, wk, wv, g, b)


def _tc_final(h, accf, ws):
    acc = accf.reshape(4, AROWS, W)

    def body(h_ref, acc_ref, ws_ref, out_ref):
        o = _combine_acc(acc_ref) + jnp.dot(h_ref[...], ws_ref[...], **_DOT)
        out_ref[...] = _gelu(o)

    return pl.pallas_call(
        body,
        grid=(N // _BN,),
        in_specs=[pl.BlockSpec((_BN, D), lambda i: (i, 0)),
                  pl.BlockSpec((4, _BN, W), lambda i: (0, i, 0)),
                  pl.BlockSpec((D, D), lambda i: (0, 0))],
        out_specs=pl.BlockSpec((_BN, D), lambda i: (i, 0)),
        out_shape=jax.ShapeDtypeStruct((N, D), _f32),
    )(h, acc, ws)


def kernel(x, edge_index, Wq, Wk, Wv, Ws, param_multi, ln_g, ln_b):
    src = edge_index[0].astype(jnp.int32)
    dst = edge_index[1].astype(jnp.int32)
    pm16 = jnp.concatenate([param_multi.astype(_f32), jnp.zeros((8,), _f32)])

    q0, k0, v4 = _tc_project0(x, Wq[0], Wk[0], Wv[0])
    accf, gate, _ = _sc_layer0(q0, k0, v4.reshape(4 * N, 64), src, dst, pm16)

    h = x
    for i in range(1, 5):
        h, q4, k4, v4 = _tc_mid(h, accf, Ws[i - 1], Wq[i], Wk[i], Wv[i],
                                ln_g[i - 1][None, :], ln_b[i - 1][None, :])
        accf = _sc_layer(q4.reshape(4 * N, 64), k4.reshape(4 * N, 64),
                         v4.reshape(4 * N, 64), src, dst, gate)

    return _tc_final(h, accf, Ws[4])
